# baseline (speedup 1.0000x reference)
import jax
import jax.numpy as jnp
from jax import lax
from jax.experimental import pallas as pl
from jax.experimental.pallas import tpu as pltpu
from jax.experimental.pallas import tpu_sc as plsc

NUM_USER_K = 100000
NUM_ITEM_K = 100000
HIDDEN_K = 128
BATCH_K = 1024
SCALE_K = 1.0 / 16.0

_NC = 2
_NS = 16
_NW = _NC * _NS
_B_PER_W = BATCH_K // _NW


def _sc_gather_body(table_hbm, idx_hbm, out_hbm, idx_v, rows_v, sem):
    wid = lax.axis_index("s") * _NC + lax.axis_index("c")
    base = wid * _B_PER_W
    pltpu.sync_copy(idx_hbm.at[pl.ds(base, _B_PER_W)], idx_v)
    pltpu.async_copy(table_hbm.at[idx_v], rows_v, sem).wait()
    pltpu.sync_copy(rows_v, out_hbm.at[pl.ds(base, _B_PER_W)])


def _sc_gather(user_emb, input_idx):
    k = pl.kernel(
        _sc_gather_body,
        mesh=plsc.VectorSubcoreMesh(core_axis_name="c", subcore_axis_name="s"),
        out_type=jax.ShapeDtypeStruct((BATCH_K, HIDDEN_K), jnp.float32),
        scratch_types=[
            pltpu.VMEM((_B_PER_W,), jnp.int32),
            pltpu.VMEM((_B_PER_W, HIDDEN_K), jnp.float32),
            pltpu.SemaphoreType.DMA,
        ],
    )
    return k(user_emb, input_idx)


_BN = 4096
_NSTEP = (NUM_ITEM_K + _BN - 1) // _BN
_TAIL = NUM_ITEM_K - (_NSTEP - 1) * _BN


def _mm_body(a_ref, b_ref, o_ref):
    o_ref[...] = SCALE_K * lax.dot_general(
        b_ref[...], a_ref[...],
        dimension_numbers=(((1,), (1,)), ((), ())),
        preferred_element_type=jnp.float32,
    )


def _matmul_t(user_batch, item_emb):
    return pl.pallas_call(
        _mm_body,
        grid=(_NSTEP,),
        in_specs=[
            pl.BlockSpec((BATCH_K, HIDDEN_K), lambda i: (0, 0)),
            pl.BlockSpec((_BN, HIDDEN_K), lambda i: (i, 0)),
        ],
        out_specs=pl.BlockSpec((_BN, BATCH_K), lambda i: (i, 0)),
        out_shape=jax.ShapeDtypeStruct((NUM_ITEM_K, BATCH_K), jnp.float32),
    )(user_batch, item_emb)


def _fill_body(c_hbm, zbuf, zsem):
    zbuf[...] = jnp.zeros((_BN, BATCH_K), jnp.float32)

    def _zcopy_full(j):
        return pltpu.make_async_copy(
            zbuf, c_hbm.at[pl.ds(j * _BN, _BN)], zsem
        )

    for j in range(_NSTEP - 1):
        _zcopy_full(j).start()
    tail = pltpu.make_async_copy(
        zbuf.at[pl.ds(0, _TAIL)],
        c_hbm.at[pl.ds((_NSTEP - 1) * _BN, _TAIL)],
        zsem,
    )
    tail.start()
    for _ in range(_NSTEP - 1):
        _zcopy_full(0).wait()
    tail.wait()


def _fill_zeros_t():
    return pl.pallas_call(
        _fill_body,
        out_specs=pl.BlockSpec(memory_space=pl.ANY),
        out_shape=jax.ShapeDtypeStruct((NUM_ITEM_K, BATCH_K), jnp.float32),
        scratch_shapes=[
            pltpu.VMEM((_BN, BATCH_K), jnp.float32),
            pltpu.SemaphoreType.DMA,
        ],
    )()


@jax.jit
def kernel(input, input_idx, user_emb, item_emb):
    del input
    user_batch = _sc_gather(user_emb, input_idx.astype(jnp.int32))
    c_t = _fill_zeros_t()
    user_batch, c_t = lax.optimization_barrier((user_batch, c_t))
    output_t = _matmul_t(user_batch, item_emb)
    return (output_t.T, c_t.T)


# speedup vs baseline: 1.0213x; 1.0213x over previous
import jax
import jax.numpy as jnp
from jax import lax
from jax.experimental import pallas as pl
from jax.experimental.pallas import tpu as pltpu
from jax.experimental.pallas import tpu_sc as plsc

NUM_USER_K = 100000
NUM_ITEM_K = 100000
HIDDEN_K = 128
BATCH_K = 1024
SCALE_K = 1.0 / 16.0

_NC = 2
_NS = 16
_NW = _NC * _NS
_B_PER_W = BATCH_K // _NW


def _sc_gather_body(table_hbm, idx_hbm, out_hbm, idx_v, rows_v, sem):
    wid = lax.axis_index("s") * _NC + lax.axis_index("c")
    base = wid * _B_PER_W
    pltpu.sync_copy(idx_hbm.at[pl.ds(base, _B_PER_W)], idx_v)
    pltpu.async_copy(table_hbm.at[idx_v], rows_v, sem).wait()
    pltpu.sync_copy(rows_v, out_hbm.at[pl.ds(base, _B_PER_W)])


def _sc_gather(user_emb, input_idx):
    k = pl.kernel(
        _sc_gather_body,
        mesh=plsc.VectorSubcoreMesh(core_axis_name="c", subcore_axis_name="s"),
        out_type=jax.ShapeDtypeStruct((BATCH_K, HIDDEN_K), jnp.float32),
        scratch_types=[
            pltpu.VMEM((_B_PER_W,), jnp.int32),
            pltpu.VMEM((_B_PER_W, HIDDEN_K), jnp.float32),
            pltpu.SemaphoreType.DMA,
        ],
    )
    return k(user_emb, input_idx)


_BN = 4096
_NSTEP = (NUM_ITEM_K + _BN - 1) // _BN
_TAIL = NUM_ITEM_K - (_NSTEP - 1) * _BN


def _mm_body(a_ref, b_ref, o_ref, c_hbm, zbuf, zsem):
    i = pl.program_id(0)

    def _zcopy_full(j):
        return pltpu.make_async_copy(
            zbuf, c_hbm.at[pl.ds(j * _BN, _BN)], zsem
        )

    @pl.when(i == 0)
    def _():
        zbuf[...] = jnp.zeros((_BN, BATCH_K), jnp.float32)

    @pl.when(i < _NSTEP - 1)
    def _():
        _zcopy_full(i).start()

    o_ref[...] = SCALE_K * lax.dot_general(
        b_ref[...], a_ref[...],
        dimension_numbers=(((1,), (1,)), ((), ())),
        preferred_element_type=jnp.float32,
    )

    @pl.when(i == _NSTEP - 1)
    def _():
        tail = pltpu.make_async_copy(
            zbuf.at[pl.ds(0, _TAIL)],
            c_hbm.at[pl.ds((_NSTEP - 1) * _BN, _TAIL)],
            zsem,
        )
        tail.start()
        for _ in range(_NSTEP - 1):
            _zcopy_full(0).wait()
        tail.wait()


def _matmul_t(user_batch, item_emb):
    return pl.pallas_call(
        _mm_body,
        grid=(_NSTEP,),
        in_specs=[
            pl.BlockSpec((BATCH_K, HIDDEN_K), lambda i: (0, 0)),
            pl.BlockSpec((_BN, HIDDEN_K), lambda i: (i, 0)),
        ],
        out_specs=[
            pl.BlockSpec((_BN, BATCH_K), lambda i: (i, 0)),
            pl.BlockSpec(memory_space=pl.ANY),
        ],
        out_shape=[
            jax.ShapeDtypeStruct((NUM_ITEM_K, BATCH_K), jnp.float32),
            jax.ShapeDtypeStruct((NUM_ITEM_K, BATCH_K), jnp.float32),
        ],
        scratch_shapes=[
            pltpu.VMEM((_BN, BATCH_K), jnp.float32),
            pltpu.SemaphoreType.DMA,
        ],
    )(user_batch, item_emb)


@jax.jit
def kernel(input, input_idx, user_emb, item_emb):
    del input
    user_batch = _sc_gather(user_emb, input_idx.astype(jnp.int32))
    output_t, c_t = _matmul_t(user_batch, item_emb)
    return (output_t.T, c_t.T)
